# baseline (speedup 1.0000x reference)
import functools

import jax
import jax.numpy as jnp
from jax import lax
from jax.experimental import pallas as pl
from jax.experimental.pallas import tpu as pltpu

try:
    from jax.experimental.pallas import tpu_sc as plsc
except ImportError:
    plsc = None

def _median_body(x_ref, o_ref, *, n, rows):
    nc = 2 * n
    half = n // 2
    _I32_MIN = jnp.int32(-(2**31))
    _I32_MAX = jnp.int32(2**31 - 1)
    _SIGNMASK = jnp.int32(0x7FFFFFFF)
    x = x_ref[...].reshape(rows, nc)
    b = jax.lax.bitcast_convert_type(x, jnp.int32)
    s = jnp.where(b < 0, b ^ _SIGNMASK, b)
    lane = jax.lax.broadcasted_iota(jnp.int32, (rows, nc), 1)
    evi = jnp.where(lane % 2 == 0, jnp.int32(1), jnp.int32(0))
    ev = lane % 2 == 0

    def bcast2(v):
        c0 = jnp.broadcast_to(v[:, 0:1], (rows, nc))
        c1 = jnp.broadcast_to(v[:, 1:2], (rows, nc))
        return jnp.where(ev, c0, c1)

    payload = jnp.where(ev, jnp.int32(1), jnp.int32(1 << 16))

    def parity_counts(cmp):
        packed = jnp.sum(jnp.where(cmp, payload, jnp.int32(0)),
                         axis=1, keepdims=True)
        c0 = packed & jnp.int32(0xFFFF)
        c1 = jax.lax.shift_right_logical(packed, 16)
        return jnp.concatenate([c0, c1], axis=1)

    def step(i, p_u):
        bit = jax.lax.shift_left(jnp.int32(1), 31 - i)
        cand_u = p_u | bit
        cand_s = cand_u ^ _I32_MIN
        cnts = parity_counts(s < bcast2(cand_s))
        take = cnts <= jnp.int32(half - 1)
        return jnp.where(take, cand_u, p_u)

    p_u = jax.lax.fori_loop(0, 32, step, jnp.zeros((rows, 2), jnp.int32))
    s_lo = p_u ^ _I32_MIN

    le = s <= bcast2(s_lo)
    c_le = parity_counts(le)
    s_ab = jnp.where(le, _I32_MAX, s)
    m0 = jnp.min(jnp.where(ev, s_ab, _I32_MAX), axis=1, keepdims=True)
    m1 = jnp.min(jnp.where(ev, _I32_MAX, s_ab), axis=1, keepdims=True)
    m_above = jnp.concatenate([m0, m1], axis=1)
    s_hi = jnp.where(c_le >= jnp.int32(half + 1), s_lo, m_above)

    def to_f32(sk):
        return jax.lax.bitcast_convert_type(
            jnp.where(sk < 0, sk ^ _SIGNMASK, sk), jnp.float32)

    o_ref[...] = (to_f32(s_lo) + to_f32(s_hi)) * jnp.float32(0.5)


def _sc_median(x2d, b0, nb, n):
    nc = 2 * n
    half = n // 2
    info = plsc.get_sparse_core_info()
    NC = info.num_cores
    NW = NC * info.num_subcores
    rows_per = nb // NW
    chunk_rows = 8
    nchunks = rows_per // chunk_rows
    chunk_elems = chunk_rows * nc
    slices_row = nc // 16
    slices_chunk = chunk_elems // 16
    mesh = plsc.VectorSubcoreMesh(core_axis_name="c", subcore_axis_name="s")

    @functools.partial(
        pl.kernel, mesh=mesh,
        out_type=jax.ShapeDtypeStruct((NW, 2 * chunk_rows * 2 * nchunks),
                                      jnp.int32),
        scratch_types=[
            pltpu.VMEM((chunk_rows, nc), jnp.int32),
            pltpu.VMEM((chunk_rows, nc), jnp.int32),
            pltpu.VMEM((chunk_rows, nc), jnp.int32),
            pltpu.VMEM((2 * chunk_rows * 2 * nchunks,), jnp.int32),
            pltpu.SemaphoreType.DMA,
            pltpu.SemaphoreType.DMA,
        ],
    )
    def sc_kern(x_hbm, out_hbm, in0, in1, keys, res, sem0, sem1):
        _I32_MIN = jnp.int32(-(2**31))
        _I32_MAX = jnp.int32(2**31 - 1)
        _SIGNMASK = jnp.int32(0x7FFFFFFF)
        wid = lax.axis_index("s") * NC + lax.axis_index("c")
        base = b0 + wid * rows_per

        lane = lax.broadcasted_iota(jnp.int32, (16,), 0)
        pairid = lax.shift_right_logical(lane, 1)
        gdn = lax.GatherDimensionNumbers(
            offset_dims=(), collapsed_slice_dims=(0,), start_index_map=(0,))

        def shuf(v, d):
            return lax.gather(
                v, (lane ^ d).reshape(16, 1), gdn, (1,),
                mode=lax.GatherScatterMode.PROMISE_IN_BOUNDS)

        def parity_allreduce(v, op):
            for d in (2, 4, 8):
                v = op(v, shuf(v, d))
            return v

        def chunk_src(cc):
            return x_hbm.at[pl.ds(base + cc * chunk_rows, chunk_rows)]

        ins = (in0, in1)
        sems = (sem0, sem1)
        pltpu.make_async_copy(chunk_src(0), in0, sem0).start()
        pltpu.make_async_copy(chunk_src(1), in1, sem1).start()

        def do_chunk(cc, in_ref, sem):
            pltpu.make_async_copy(chunk_src(cc), in_ref, sem).wait()

            def tf_row(rr, _):
                def tf(j, _2):
                    for u in range(16):
                        off = j * 256 + u * 16
                        k = in_ref[rr, pl.ds(off, 16)]
                        keys[rr, pl.ds(off, 16)] = jnp.where(
                            k < 0, k ^ _SIGNMASK, k)
                    return jnp.int32(0)

                return lax.fori_loop(0, slices_row // 16, tf, jnp.int32(0))

            lax.fori_loop(0, chunk_rows, tf_row, jnp.int32(0))

            @pl.when(cc + 2 < nchunks)
            def _():
                pltpu.make_async_copy(chunk_src(cc + 2), in_ref, sem).start()

            def row_fn(rr, res_vecs):
                def bitstep(i, p_vec):
                    bit = lax.shift_left(jnp.int32(1), jnp.int32(31) - i)
                    cand_u = p_vec | bit
                    cand = cand_u ^ _I32_MIN

                    def cnt_step(j, cnt):
                        for u in range(16):
                            blk = keys[rr, pl.ds(j * 256 + u * 16, 16)]
                            cnt = cnt + jnp.where(blk < cand, jnp.int32(1),
                                                  jnp.int32(0))
                        return cnt

                    cnt = lax.fori_loop(0, slices_row // 16, cnt_step,
                                        jnp.zeros((16,), jnp.int32))
                    cnt = parity_allreduce(cnt, jnp.add)
                    return jnp.where(cnt <= jnp.int32(half - 1),
                                     cand_u, p_vec)

                p_vec = lax.fori_loop(0, 32, bitstep,
                                      jnp.zeros((16,), jnp.int32))
                slo_vec = p_vec ^ _I32_MIN

                def fin(j, carry):
                    cnt, mins = carry
                    for u in range(16):
                        blk = keys[rr, pl.ds(j * 256 + u * 16, 16)]
                        le = blk <= slo_vec
                        cnt = cnt + jnp.where(le, jnp.int32(1), jnp.int32(0))
                        mins = jnp.minimum(mins,
                                           jnp.where(le, _I32_MAX, blk))
                    return cnt, mins

                cnt, mins = lax.fori_loop(
                    0, slices_row // 16, fin,
                    (jnp.zeros((16,), jnp.int32),
                     jnp.full((16,), _I32_MAX, jnp.int32)))
                cle = parity_allreduce(cnt, jnp.add)
                mab = parity_allreduce(mins, jnp.minimum)
                shi_vec = jnp.where(cle >= jnp.int32(half + 1),
                                    slo_vec, mab)
                lo_vec, hi_vec = res_vecs
                sel = pairid == rr
                return (jnp.where(sel, slo_vec, lo_vec),
                        jnp.where(sel, shi_vec, hi_vec))

            lo_vec, hi_vec = lax.fori_loop(
                0, chunk_rows, row_fn,
                (jnp.zeros((16,), jnp.int32), jnp.zeros((16,), jnp.int32)))
            res[pl.ds(cc * 32, 16)] = lo_vec
            res[pl.ds(cc * 32 + 16, 16)] = hi_vec

        def pair_fn(p, _):
            for b in range(2):
                do_chunk(2 * p + b, ins[b], sems[b])
            return jnp.int32(0)

        lax.fori_loop(0, nchunks // 2, pair_fn, jnp.int32(0))
        if nchunks % 2:
            do_chunk(jnp.int32(nchunks - 1), ins[0], sems[0])
        pltpu.sync_copy(res, out_hbm.at[wid])

    out = sc_kern(x2d)
    r = out.reshape(NW, nchunks, 2, 16)
    lo = r[:, :, 0, :].reshape(nb, 2)
    hi = r[:, :, 1, :].reshape(nb, 2)

    def unmap(s):
        return lax.bitcast_convert_type(
            jnp.where(s < 0, s ^ jnp.int32(0x7FFFFFFF), s), jnp.float32)

    return (unmap(lo) + unmap(hi)) * jnp.float32(0.5)


def _tc_median(x2d, b, n):
    rows = 128 if b % 128 == 0 else 8
    body = functools.partial(_median_body, n=n, rows=rows)
    return pl.pallas_call(
        body,
        grid=(b // rows,),
        in_specs=[pl.BlockSpec((rows, n * 2), lambda i: (i, 0))],
        out_specs=pl.BlockSpec((rows, 2), lambda i: (i, 0)),
        out_shape=jax.ShapeDtypeStruct((b, 2), jnp.float32),
    )(x2d)


def kernel(inputs):
    b, n, c = inputs.shape
    assert c == 2 and n % 2 == 0
    b_sc = (b * 7 // 16) // 256 * 256 if plsc is not None else 0
    x2d = inputs.reshape(b, n * c)
    if b_sc and (b - b_sc) % 128 == 0:
        x_i = lax.bitcast_convert_type(x2d, jnp.int32)
        med_sc = _sc_median(x_i, b - b_sc, b_sc, n)
        med_tc = _tc_median(x2d, b - b_sc, n)
        med = jnp.concatenate([med_tc, med_sc], axis=0)
    else:
        med = _tc_median(x2d, b, n)
    return med.reshape(b, 1, c)


# speedup vs baseline: 1.0396x; 1.0396x over previous
import functools

import jax
import jax.numpy as jnp
from jax import lax
from jax.experimental import pallas as pl
from jax.experimental.pallas import tpu as pltpu

try:
    from jax.experimental.pallas import tpu_sc as plsc
except ImportError:
    plsc = None

def _median_body(x_ref, o_ref, *, n, rows):
    nc = 2 * n
    half = n // 2
    _I32_MIN = jnp.int32(-(2**31))
    _I32_MAX = jnp.int32(2**31 - 1)
    _SIGNMASK = jnp.int32(0x7FFFFFFF)
    x = x_ref[...].reshape(rows, nc)
    b = jax.lax.bitcast_convert_type(x, jnp.int32)
    s = jnp.where(b < 0, b ^ _SIGNMASK, b)
    lane = jax.lax.broadcasted_iota(jnp.int32, (rows, nc), 1)
    evi = jnp.where(lane % 2 == 0, jnp.int32(1), jnp.int32(0))
    ev = lane % 2 == 0

    def bcast2(v):
        c0 = jnp.broadcast_to(v[:, 0:1], (rows, nc))
        c1 = jnp.broadcast_to(v[:, 1:2], (rows, nc))
        return jnp.where(ev, c0, c1)

    payload = jnp.where(ev, jnp.int32(1), jnp.int32(1 << 16))

    def parity_counts(cmp):
        packed = jnp.sum(jnp.where(cmp, payload, jnp.int32(0)),
                         axis=1, keepdims=True)
        c0 = packed & jnp.int32(0xFFFF)
        c1 = jax.lax.shift_right_logical(packed, 16)
        return jnp.concatenate([c0, c1], axis=1)

    def step(i, p_u):
        bit = jax.lax.shift_left(jnp.int32(1), 31 - i)
        cand_u = p_u | bit
        cand_s = cand_u ^ _I32_MIN
        cnts = parity_counts(s < bcast2(cand_s))
        take = cnts <= jnp.int32(half - 1)
        return jnp.where(take, cand_u, p_u)

    p_u = jax.lax.fori_loop(0, 32, step, jnp.zeros((rows, 2), jnp.int32))
    s_lo = p_u ^ _I32_MIN

    le = s <= bcast2(s_lo)
    c_le = parity_counts(le)
    s_ab = jnp.where(le, _I32_MAX, s)
    m0 = jnp.min(jnp.where(ev, s_ab, _I32_MAX), axis=1, keepdims=True)
    m1 = jnp.min(jnp.where(ev, _I32_MAX, s_ab), axis=1, keepdims=True)
    m_above = jnp.concatenate([m0, m1], axis=1)
    s_hi = jnp.where(c_le >= jnp.int32(half + 1), s_lo, m_above)

    def to_f32(sk):
        return jax.lax.bitcast_convert_type(
            jnp.where(sk < 0, sk ^ _SIGNMASK, sk), jnp.float32)

    o_ref[...] = (to_f32(s_lo) + to_f32(s_hi)) * jnp.float32(0.5)


def _sc_median(x2d, b0, nb, n):
    nc = 2 * n
    half = n // 2
    info = plsc.get_sparse_core_info()
    NC = info.num_cores
    NW = NC * info.num_subcores
    rows_per = nb // NW
    chunk_rows = 8
    nchunks = rows_per // chunk_rows
    chunk_elems = chunk_rows * nc
    slices_row = nc // 16
    slices_chunk = chunk_elems // 16
    mesh = plsc.VectorSubcoreMesh(core_axis_name="c", subcore_axis_name="s")

    @functools.partial(
        pl.kernel, mesh=mesh,
        out_type=jax.ShapeDtypeStruct((NW, 2 * chunk_rows * 2 * nchunks),
                                      jnp.int32),
        scratch_types=[
            pltpu.VMEM((chunk_rows, nc), jnp.int32),
            pltpu.VMEM((chunk_rows, nc), jnp.int32),
            pltpu.VMEM((chunk_rows, nc), jnp.int32),
            pltpu.VMEM((2 * chunk_rows * 2 * nchunks,), jnp.int32),
            pltpu.SemaphoreType.DMA,
            pltpu.SemaphoreType.DMA,
        ],
    )
    def sc_kern(x_hbm, out_hbm, in0, in1, keys, res, sem0, sem1):
        _I32_MIN = jnp.int32(-(2**31))
        _I32_MAX = jnp.int32(2**31 - 1)
        _SIGNMASK = jnp.int32(0x7FFFFFFF)
        wid = lax.axis_index("s") * NC + lax.axis_index("c")
        base = b0 + wid * rows_per

        lane = lax.broadcasted_iota(jnp.int32, (16,), 0)
        pairid = lax.shift_right_logical(lane, 1)
        gdn = lax.GatherDimensionNumbers(
            offset_dims=(), collapsed_slice_dims=(0,), start_index_map=(0,))

        def shuf(v, d):
            return lax.gather(
                v, (lane ^ d).reshape(16, 1), gdn, (1,),
                mode=lax.GatherScatterMode.PROMISE_IN_BOUNDS)

        def parity_allreduce(v, op):
            for d in (2, 4, 8):
                v = op(v, shuf(v, d))
            return v

        def chunk_src(cc):
            return x_hbm.at[pl.ds(base + cc * chunk_rows, chunk_rows)]

        ins = (in0, in1)
        sems = (sem0, sem1)
        pltpu.make_async_copy(chunk_src(0), in0, sem0).start()
        pltpu.make_async_copy(chunk_src(1), in1, sem1).start()

        def do_chunk(cc, in_ref, sem):
            pltpu.make_async_copy(chunk_src(cc), in_ref, sem).wait()

            def tf_row(rr, _):
                def tf(j, _2):
                    for u in range(16):
                        off = j * 256 + u * 16
                        k = in_ref[rr, pl.ds(off, 16)]
                        keys[rr, pl.ds(off, 16)] = jnp.where(
                            k < 0, k ^ _SIGNMASK, k)
                    return jnp.int32(0)

                return lax.fori_loop(0, slices_row // 16, tf, jnp.int32(0))

            lax.fori_loop(0, chunk_rows, tf_row, jnp.int32(0))

            @pl.when(cc + 2 < nchunks)
            def _():
                pltpu.make_async_copy(chunk_src(cc + 2), in_ref, sem).start()

            def row_fn(rr, res_vecs):
                def bitstep(i, p_vec):
                    bit = lax.shift_left(jnp.int32(1), jnp.int32(31) - i)
                    cand_u = p_vec | bit
                    cand = cand_u ^ _I32_MIN

                    def cnt_step(j, cnt):
                        for u in range(16):
                            blk = keys[rr, pl.ds(j * 256 + u * 16, 16)]
                            cnt = cnt + jnp.where(blk < cand, jnp.int32(1),
                                                  jnp.int32(0))
                        return cnt

                    cnt = lax.fori_loop(0, slices_row // 16, cnt_step,
                                        jnp.zeros((16,), jnp.int32))
                    cnt = parity_allreduce(cnt, jnp.add)
                    return jnp.where(cnt <= jnp.int32(half - 1),
                                     cand_u, p_vec)

                p_vec = lax.fori_loop(0, 32, bitstep,
                                      jnp.zeros((16,), jnp.int32))
                slo_vec = p_vec ^ _I32_MIN

                def fin(j, carry):
                    cnt, mins = carry
                    for u in range(16):
                        blk = keys[rr, pl.ds(j * 256 + u * 16, 16)]
                        le = blk <= slo_vec
                        cnt = cnt + jnp.where(le, jnp.int32(1), jnp.int32(0))
                        mins = jnp.minimum(mins,
                                           jnp.where(le, _I32_MAX, blk))
                    return cnt, mins

                cnt, mins = lax.fori_loop(
                    0, slices_row // 16, fin,
                    (jnp.zeros((16,), jnp.int32),
                     jnp.full((16,), _I32_MAX, jnp.int32)))
                cle = parity_allreduce(cnt, jnp.add)
                mab = parity_allreduce(mins, jnp.minimum)
                shi_vec = jnp.where(cle >= jnp.int32(half + 1),
                                    slo_vec, mab)
                lo_vec, hi_vec = res_vecs
                sel = pairid == rr
                return (jnp.where(sel, slo_vec, lo_vec),
                        jnp.where(sel, shi_vec, hi_vec))

            lo_vec, hi_vec = lax.fori_loop(
                0, chunk_rows, row_fn,
                (jnp.zeros((16,), jnp.int32), jnp.zeros((16,), jnp.int32)))
            res[pl.ds(cc * 32, 16)] = lo_vec
            res[pl.ds(cc * 32 + 16, 16)] = hi_vec

        def pair_fn(p, _):
            for b in range(2):
                do_chunk(2 * p + b, ins[b], sems[b])
            return jnp.int32(0)

        lax.fori_loop(0, nchunks // 2, pair_fn, jnp.int32(0))
        if nchunks % 2:
            do_chunk(jnp.int32(nchunks - 1), ins[0], sems[0])
        pltpu.sync_copy(res, out_hbm.at[wid])

    out = sc_kern(x2d)
    r = out.reshape(NW, nchunks, 2, 16)
    lo = r[:, :, 0, :].reshape(nb, 2)
    hi = r[:, :, 1, :].reshape(nb, 2)

    def unmap(s):
        return lax.bitcast_convert_type(
            jnp.where(s < 0, s ^ jnp.int32(0x7FFFFFFF), s), jnp.float32)

    return (unmap(lo) + unmap(hi)) * jnp.float32(0.5)


def _tc_median(x2d, b, n):
    rows = 128 if b % 128 == 0 else 8
    body = functools.partial(_median_body, n=n, rows=rows)
    return pl.pallas_call(
        body,
        grid=(b // rows,),
        in_specs=[pl.BlockSpec((rows, n * 2), lambda i: (i, 0))],
        out_specs=pl.BlockSpec((rows, 2), lambda i: (i, 0)),
        out_shape=jax.ShapeDtypeStruct((b, 2), jnp.float32),
    )(x2d)


def kernel(inputs):
    b, n, c = inputs.shape
    assert c == 2 and n % 2 == 0
    b_sc = (b * 3 // 8) // 256 * 256 if plsc is not None else 0
    x2d = inputs.reshape(b, n * c)
    if b_sc and (b - b_sc) % 128 == 0:
        x_i = lax.bitcast_convert_type(x2d, jnp.int32)
        med_sc = _sc_median(x_i, b - b_sc, b_sc, n)
        med_tc = _tc_median(x2d, b - b_sc, n)
        med = jnp.concatenate([med_tc, med_sc], axis=0)
    else:
        med = _tc_median(x2d, b, n)
    return med.reshape(b, 1, c)
